# baseline (speedup 1.0000x reference)
import jax
import jax.numpy as jnp
from jax import lax
from jax.experimental import pallas as pl
from jax.experimental.pallas import tpu as pltpu
from jax.experimental.pallas import tpu_sc as plsc

_H = 32
_N = 64


def _bias_body(tabs_hbm, out_hbm, tabs_v, out_v, sem_t, sem_o):
    wid = lax.axis_index("s")

    cp_t = pltpu.async_copy(tabs_hbm, tabs_v, sem_t)

    lane = lax.broadcasted_iota(jnp.int32, (16,), 0)

    cp_t.wait()
    rt_tab = tabs_v.at[0]
    ct_tab = tabs_v.at[1]

    def build_plane(hh):
        colsel = jnp.full((16,), hh, jnp.int32) + 2 * wid

        cvec = [plsc.load_gather(ct_tab, [p + 7 - (lane & 7), colsel])
                for p in range(8)]

        @plsc.parallel_loop(0, 8, unroll=2)
        def block_body(a):
            rvec = [
                plsc.load_gather(
                    rt_tab, [a + 7 - ((c * 16 + lane) >> 3), colsel])
                for c in range(4)
            ]
            for p in range(8):
                for c in range(4):
                    out_v[hh, a * 8 + p, pl.ds(c * 16, 16)] = rvec[c] + cvec[p]

    build_plane(0)
    build_plane(1)
    pltpu.sync_copy(out_v, out_hbm.at[0, pl.ds(2 * wid, 2)])


@jax.jit
def _bias_planes(row_table, col_table):
    tabs = jnp.stack([row_table, col_table])
    mesh = plsc.VectorSubcoreMesh(
        core_axis_name="c", subcore_axis_name="s", num_cores=1)
    return pl.kernel(
        _bias_body,
        mesh=mesh,
        out_type=jax.ShapeDtypeStruct((1, _H, _N, _N), jnp.float32),
        scratch_types=[
            pltpu.VMEM((2, 15, _H), jnp.float32),
            pltpu.VMEM((2, _N, _N), jnp.float32),
            pltpu.SemaphoreType.DMA,
            pltpu.SemaphoreType.DMA,
        ],
        compiler_params=pltpu.CompilerParams(
            needs_layout_passes=False,
            disable_bounds_checks=True,
            skip_device_barrier=True,
        ),
    )(tabs)


def kernel(q_len, k_len, row_bias_table, col_bias_table):
    return _bias_planes(row_bias_table, col_bias_table)


# speedup vs baseline: 1.0151x; 1.0151x over previous
import jax
import jax.numpy as jnp
from jax import lax
from jax.experimental import pallas as pl
from jax.experimental.pallas import tpu as pltpu
from jax.experimental.pallas import tpu_sc as plsc

_H = 32
_N = 64


def _bias_body(tabs_hbm, out_hbm, tabs_v, out_v, sem_t, sem_o):
    wid = lax.axis_index("s")

    cp_t = pltpu.async_copy(tabs_hbm, tabs_v, sem_t)

    lane = lax.broadcasted_iota(jnp.int32, (16,), 0)

    cp_t.wait()
    rt_tab = tabs_v.at[0]
    ct_tab = tabs_v.at[1]

    def build_plane(hh):
        colsel = jnp.full((16,), hh, jnp.int32) + 2 * wid

        cvec = [plsc.load_gather(ct_tab, [p + 7 - (lane & 7), colsel])
                for p in range(8)]

        @plsc.parallel_loop(0, 8, unroll=2)
        def block_body(a):
            rvec = [
                plsc.load_gather(
                    rt_tab, [a + 7 - ((c * 16 + lane) >> 3), colsel])
                for c in range(4)
            ]
            for p in range(8):
                for c in range(4):
                    out_v[hh, a * 8 + p, pl.ds(c * 16, 16)] = rvec[c] + cvec[p]

    build_plane(0)
    cp0 = pltpu.async_copy(out_v.at[0], out_hbm.at[0, 2 * wid], sem_o)
    build_plane(1)
    cp1 = pltpu.async_copy(out_v.at[1], out_hbm.at[0, 2 * wid + 1], sem_o)
    cp0.wait()
    cp1.wait()


@jax.jit
def _bias_planes(row_table, col_table):
    tabs = jnp.stack([row_table, col_table])
    mesh = plsc.VectorSubcoreMesh(
        core_axis_name="c", subcore_axis_name="s", num_cores=1)
    return pl.kernel(
        _bias_body,
        mesh=mesh,
        out_type=jax.ShapeDtypeStruct((1, _H, _N, _N), jnp.float32),
        scratch_types=[
            pltpu.VMEM((2, 15, _H), jnp.float32),
            pltpu.VMEM((2, _N, _N), jnp.float32),
            pltpu.SemaphoreType.DMA,
            pltpu.SemaphoreType.DMA,
        ],
        compiler_params=pltpu.CompilerParams(
            needs_layout_passes=False,
            disable_bounds_checks=True,
            skip_device_barrier=True,
        ),
    )(tabs)


def kernel(q_len, k_len, row_bias_table, col_bias_table):
    return _bias_planes(row_bias_table, col_bias_table)
